# Initial kernel scaffold; baseline (speedup 1.0000x reference)
import jax
import jax.numpy as jnp
from jax.experimental import pallas as pl


def kernel(x, edge_index, W1, b1, W2, b2):
    raise NotImplementedError("write your pallas kernel here")


# speedup vs baseline: 7.4591x; 7.4591x over previous
import functools

import jax
import jax.numpy as jnp
from jax import lax
from jax.experimental import pallas as pl
from jax.experimental.pallas import tpu as pltpu
from jax.experimental.pallas import tpu_sc as plsc

_N = 10000
_NP = 10240
_HALF = _NP // 2
_E = 320000
_EP = 327680
_PAD = _EP - _E
_CH = 160
_NBUF = 2
_STRIPE = _HALF // 16
_B = 1024
_IGN = -1


def _mesh():
    return plsc.VectorSubcoreMesh(core_axis_name="c", subcore_axis_name="s")


def _sc_hist(dst_h):
    @functools.partial(
        pl.kernel,
        out_type=jax.ShapeDtypeStruct((_NP,), jnp.float32),
        mesh=_mesh(),
        scratch_types=[
            pltpu.VMEM((_CH, 128), jnp.int32),
            pltpu.VMEM((128,), jnp.float32),
            pltpu.VMEM((_STRIPE,), jnp.float32),
            pltpu.VMEM_SHARED((_HALF,), jnp.float32),
            pltpu.SemaphoreType.DMA,
        ],
    )
    def k(dst_ref, out_ref, didx, ones, zbuf, hist, sem):
        c = lax.axis_index("c")
        s = lax.axis_index("s")
        w = c * 16 + s
        ov = jnp.ones((16,), jnp.float32)
        zv = jnp.zeros((16,), jnp.float32)

        def fill(i, _):
            ones[pl.ds(i * 16, 16)] = ov
            return 0

        lax.fori_loop(0, 8, fill, 0)

        def zrow(i, _):
            zbuf[pl.ds(i * 16, 16)] = zv
            return 0

        lax.fori_loop(0, _STRIPE // 16, zrow, 0)
        pltpu.sync_copy(zbuf, hist.at[pl.ds(s * _STRIPE, _STRIPE)])
        pltpu.sync_copy(dst_ref.at[w], didx)
        plsc.subcore_barrier()

        def body(t, _):
            descs = []
            for b in range(8):
                descs.append(
                    pltpu.async_copy(
                        ones,
                        hist.at[
                            plsc.Indices(
                                didx.at[t * 8 + b], ignored_value=_IGN
                            )
                        ],
                        sem,
                        add=True,
                    )
                )
            for d in descs:
                d.wait()
            return 0

        lax.fori_loop(0, _CH // 8, body, 0)
        plsc.subcore_barrier()
        pltpu.sync_copy(hist.at[pl.ds(s * _STRIPE, _STRIPE)], zbuf)
        pltpu.sync_copy(
            zbuf, out_ref.at[pl.ds(c * _HALF + s * _STRIPE, _STRIPE)]
        )

    return k(dst_h)


def _hist_to_col(hist):
    return hist.reshape(_NP, 1)


def _sc_agg(table, src_h, dst_h):
    @functools.partial(
        pl.kernel,
        out_type=jax.ShapeDtypeStruct((_NP, 128), jnp.float32),
        mesh=_mesh(),
        scratch_types=[
            pltpu.VMEM((_CH, 128), jnp.int32),
            pltpu.VMEM((_CH, 128), jnp.int32),
            pltpu.VMEM((_NBUF, 128, 128), jnp.float32),
            pltpu.VMEM((32, 128), jnp.float32),
            pltpu.VMEM_SHARED((_HALF, 128), jnp.float32),
            pltpu.SemaphoreType.DMA,
            pltpu.SemaphoreType.DMA,
        ],
    )
    def k(table_ref, src_ref, dst_ref, out_ref, sidx, didx, bufs, zbuf,
          accum, gsem, ssem):
        c = lax.axis_index("c")
        s = lax.axis_index("s")
        w = c * 16 + s
        zv = jnp.zeros((16,), jnp.float32)

        def zrow(i, _):
            for q in range(8):
                zbuf[i, pl.ds(q * 16, 16)] = zv
            return 0

        lax.fori_loop(0, 32, zrow, 0)

        def zcp(t, _):
            pltpu.sync_copy(zbuf, accum.at[pl.ds(s * _STRIPE + t * 32, 32)])
            return 0

        lax.fori_loop(0, _STRIPE // 32, zcp, 0)
        pltpu.sync_copy(src_ref.at[w], sidx)
        pltpu.sync_copy(dst_ref.at[w], didx)
        plsc.subcore_barrier()

        def body(t, _):
            base = t * _NBUF
            gds = []
            for b in range(_NBUF):
                gds.append(
                    pltpu.async_copy(
                        table_ref.at[
                            plsc.Indices(
                                sidx.at[base + b], ignored_value=_IGN
                            )
                        ],
                        bufs.at[b],
                        gsem,
                    )
                )
            sds = []
            for b in range(_NBUF):
                gds[b].wait()
                sds.append(
                    pltpu.async_copy(
                        bufs.at[b],
                        accum.at[
                            plsc.Indices(
                                didx.at[base + b], ignored_value=_IGN
                            )
                        ],
                        ssem,
                        add=True,
                    )
                )
            for d in sds:
                d.wait()
            return 0

        lax.fori_loop(0, _CH // _NBUF, body, 0)
        plsc.subcore_barrier()
        pltpu.sync_copy(
            accum.at[pl.ds(s * _STRIPE, _STRIPE)],
            out_ref.at[pl.ds(c * _HALF + s * _STRIPE, _STRIPE)],
        )

    return k(table, src_h, dst_h)


def _tc1(x_pad, hist, W1):
    def body(x_ref, h_ref, w_ref, hp_ref, dis_ref):
        deg = 1.0 + h_ref[...]
        dis = lax.rsqrt(deg)
        hp_ref[...] = jnp.dot(x_ref[...], w_ref[...],
                              preferred_element_type=jnp.float32) * dis
        dis_ref[...] = dis

    return pl.pallas_call(
        body,
        grid=(_NP // _B,),
        in_specs=[
            pl.BlockSpec((_B, 128), lambda i: (i, 0)),
            pl.BlockSpec((_B, 1), lambda i: (i, 0)),
            pl.BlockSpec((128, 128), lambda i: (0, 0)),
        ],
        out_specs=[
            pl.BlockSpec((_B, 128), lambda i: (i, 0)),
            pl.BlockSpec((_B, 1), lambda i: (i, 0)),
        ],
        out_shape=[
            jax.ShapeDtypeStruct((_NP, 128), jnp.float32),
            jax.ShapeDtypeStruct((_NP, 1), jnp.float32),
        ],
    )(x_pad, hist, W1)


def _tc2(a, hp, dis, W2, b1):
    def body(a_ref, h_ref, dis_ref, w_ref, b_ref, g_ref):
        t = a_ref[...] + h_ref[...]
        dis = dis_ref[...]
        z = t * dis + b_ref[...]
        h2 = jnp.maximum(z, 0.0)
        g_ref[...] = jnp.dot(h2, w_ref[...],
                             preferred_element_type=jnp.float32) * dis

    return pl.pallas_call(
        body,
        grid=(_NP // _B,),
        in_specs=[
            pl.BlockSpec((_B, 128), lambda i: (i, 0)),
            pl.BlockSpec((_B, 128), lambda i: (i, 0)),
            pl.BlockSpec((_B, 1), lambda i: (i, 0)),
            pl.BlockSpec((128, 128), lambda i: (0, 0)),
            pl.BlockSpec((1, 128), lambda i: (0, 0)),
        ],
        out_specs=pl.BlockSpec((_B, 128), lambda i: (i, 0)),
        out_shape=jax.ShapeDtypeStruct((_NP, 128), jnp.float32),
    )(a, hp, dis, W2, b1)


def _tc3(cacc, g, dis, b2):
    def body(c_ref, g_ref, dis_ref, b_ref, o_ref):
        t = c_ref[...] + g_ref[...]
        o_ref[...] = t * dis_ref[...] + b_ref[...]

    return pl.pallas_call(
        body,
        grid=(_NP // _B,),
        in_specs=[
            pl.BlockSpec((_B, 128), lambda i: (i, 0)),
            pl.BlockSpec((_B, 128), lambda i: (i, 0)),
            pl.BlockSpec((_B, 1), lambda i: (i, 0)),
            pl.BlockSpec((1, 128), lambda i: (0, 0)),
        ],
        out_specs=pl.BlockSpec((_B, 128), lambda i: (i, 0)),
        out_shape=jax.ShapeDtypeStruct((_NP, 128), jnp.float32),
    )(cacc, g, dis, b2)


def kernel(x, edge_index, W1, b1, W2, b2):
    src = edge_index[0]
    dst = edge_index[1]
    src_p = jnp.concatenate([src, jnp.zeros((_PAD,), jnp.int32)])
    dst_p = jnp.concatenate([dst, jnp.full((_PAD,), _N, jnp.int32)])
    ign = jnp.int32(_IGN)
    in0 = dst_p < _HALF
    src0 = jnp.where(in0, src_p, ign).reshape(16, _CH, 128)
    dst0 = jnp.where(in0, dst_p, ign).reshape(16, _CH, 128)
    src1 = jnp.where(in0, ign, src_p).reshape(16, _CH, 128)
    dst1 = jnp.where(in0, ign, dst_p - _HALF).reshape(16, _CH, 128)
    src_h = jnp.concatenate([src0[None], src1[None]]).reshape(32, _CH, 128)
    dst_h = jnp.concatenate([dst0[None], dst1[None]]).reshape(32, _CH, 128)
    x_pad = jnp.pad(x, ((0, _NP - _N), (0, 0)))

    hist = _hist_to_col(_sc_hist(dst_h))
    hp, dis = _tc1(x_pad, hist, W1)
    a = _sc_agg(hp, src_h, dst_h)
    g = _tc2(a, hp, dis, W2, b1.reshape(1, 128))
    cacc = _sc_agg(g, src_h, dst_h)
    out = _tc3(cacc, g, dis, b2.reshape(1, 128))
    return out[:_N]


# speedup vs baseline: 19.6487x; 2.6342x over previous
import functools

import jax
import jax.numpy as jnp
from jax import lax
from jax.experimental import pallas as pl
from jax.experimental.pallas import tpu as pltpu
from jax.experimental.pallas import tpu_sc as plsc

_N = 10000
_NP = 10240
_HALF = _NP // 2
_E = 320000
_EP = 327680
_PAD = _EP - _E
_CH = 160
_NBUF = 2
_STRIPE = _HALF // 16
_B = 1024
_IGN = -1


def _mesh():
    return plsc.VectorSubcoreMesh(core_axis_name="c", subcore_axis_name="s")


def _sc_hist(dst_h):
    @functools.partial(
        pl.kernel,
        out_type=jax.ShapeDtypeStruct((_NP,), jnp.float32),
        mesh=_mesh(),
        scratch_types=[
            pltpu.VMEM((_CH, 128), jnp.int32),
            pltpu.VMEM((128,), jnp.float32),
            pltpu.VMEM((_STRIPE,), jnp.float32),
            pltpu.VMEM_SHARED((_HALF,), jnp.float32),
            pltpu.SemaphoreType.DMA,
        ],
    )
    def k(dst_ref, out_ref, didx, ones, zbuf, hist, sem):
        c = lax.axis_index("c")
        s = lax.axis_index("s")
        w = c * 16 + s
        ov = jnp.ones((16,), jnp.float32)
        zv = jnp.zeros((16,), jnp.float32)

        def fill(i, _):
            ones[pl.ds(i * 16, 16)] = ov
            return 0

        lax.fori_loop(0, 8, fill, 0)

        def zrow(i, _):
            zbuf[pl.ds(i * 16, 16)] = zv
            return 0

        lax.fori_loop(0, _STRIPE // 16, zrow, 0)
        pltpu.sync_copy(zbuf, hist.at[pl.ds(s * _STRIPE, _STRIPE)])
        pltpu.sync_copy(dst_ref.at[w], didx)
        plsc.subcore_barrier()

        def body(t, _):
            descs = []
            for b in range(8):
                descs.append(
                    pltpu.async_copy(
                        ones,
                        hist.at[
                            plsc.Indices(
                                didx.at[t * 8 + b], ignored_value=_IGN
                            )
                        ],
                        sem,
                        add=True,
                    )
                )
            for d in descs:
                d.wait()
            return 0

        lax.fori_loop(0, _CH // 8, body, 0)
        plsc.subcore_barrier()
        pltpu.sync_copy(hist.at[pl.ds(s * _STRIPE, _STRIPE)], zbuf)
        pltpu.sync_copy(
            zbuf, out_ref.at[pl.ds(c * _HALF + s * _STRIPE, _STRIPE)]
        )

    return k(dst_h)


def _hist_to_col(hist):
    return hist.reshape(_NP, 1)


def _sc_agg(table, src_h, dst_h):
    @functools.partial(
        pl.kernel,
        out_type=jax.ShapeDtypeStruct((_NP, 128), jnp.float32),
        mesh=_mesh(),
        scratch_types=[
            pltpu.VMEM((_CH, 128), jnp.int32),
            pltpu.VMEM((_CH, 128), jnp.int32),
            pltpu.VMEM((_NBUF, 128, 128), jnp.float32),
            pltpu.VMEM((32, 128), jnp.float32),
            pltpu.VMEM_SHARED((_HALF, 128), jnp.float32),
            pltpu.SemaphoreType.DMA,
            pltpu.SemaphoreType.DMA,
        ],
    )
    def k(table_ref, src_ref, dst_ref, out_ref, sidx, didx, bufs, zbuf,
          accum, gsem, ssem):
        c = lax.axis_index("c")
        s = lax.axis_index("s")
        w = c * 16 + s
        zv = jnp.zeros((16,), jnp.float32)

        def zrow(i, _):
            for q in range(8):
                zbuf[i, pl.ds(q * 16, 16)] = zv
            return 0

        lax.fori_loop(0, 32, zrow, 0)

        def zcp(t, _):
            pltpu.sync_copy(zbuf, accum.at[pl.ds(s * _STRIPE + t * 32, 32)])
            return 0

        lax.fori_loop(0, _STRIPE // 32, zcp, 0)
        pltpu.sync_copy(src_ref.at[w], sidx)
        pltpu.sync_copy(dst_ref.at[w], didx)
        plsc.subcore_barrier()

        def body(t, _):
            base = t * _NBUF
            gds = []
            for b in range(_NBUF):
                gds.append(
                    pltpu.async_copy(
                        table_ref.at[
                            plsc.Indices(
                                sidx.at[base + b], ignored_value=_IGN
                            )
                        ],
                        bufs.at[b],
                        gsem,
                    )
                )
            sds = []
            for b in range(_NBUF):
                gds[b].wait()
                sds.append(
                    pltpu.async_copy(
                        bufs.at[b],
                        accum.at[
                            plsc.Indices(
                                didx.at[base + b], ignored_value=_IGN
                            )
                        ],
                        ssem,
                        add=True,
                    )
                )
            for d in sds:
                d.wait()
            return 0

        lax.fori_loop(0, _CH // _NBUF, body, 0)
        plsc.subcore_barrier()
        pltpu.sync_copy(
            accum.at[pl.ds(s * _STRIPE, _STRIPE)],
            out_ref.at[pl.ds(c * _HALF + s * _STRIPE, _STRIPE)],
        )

    return k(table, src_h, dst_h)


def _tc1(x_pad, hist, W1):
    def body(x_ref, h_ref, w_ref, hp_ref, dis_ref):
        deg = 1.0 + h_ref[...]
        dis = lax.rsqrt(deg)
        hp_ref[...] = jnp.dot(x_ref[...], w_ref[...],
                              preferred_element_type=jnp.float32) * dis
        dis_ref[...] = dis

    return pl.pallas_call(
        body,
        grid=(_NP // _B,),
        in_specs=[
            pl.BlockSpec((_B, 128), lambda i: (i, 0)),
            pl.BlockSpec((_B, 1), lambda i: (i, 0)),
            pl.BlockSpec((128, 128), lambda i: (0, 0)),
        ],
        out_specs=[
            pl.BlockSpec((_B, 128), lambda i: (i, 0)),
            pl.BlockSpec((_B, 1), lambda i: (i, 0)),
        ],
        out_shape=[
            jax.ShapeDtypeStruct((_NP, 128), jnp.float32),
            jax.ShapeDtypeStruct((_NP, 1), jnp.float32),
        ],
    )(x_pad, hist, W1)


def _tc2(a, hp, dis, W2, b1):
    def body(a_ref, h_ref, dis_ref, w_ref, b_ref, g_ref):
        t = a_ref[...] + h_ref[...]
        dis = dis_ref[...]
        z = t * dis + b_ref[...]
        h2 = jnp.maximum(z, 0.0)
        g_ref[...] = jnp.dot(h2, w_ref[...],
                             preferred_element_type=jnp.float32) * dis

    return pl.pallas_call(
        body,
        grid=(_NP // _B,),
        in_specs=[
            pl.BlockSpec((_B, 128), lambda i: (i, 0)),
            pl.BlockSpec((_B, 128), lambda i: (i, 0)),
            pl.BlockSpec((_B, 1), lambda i: (i, 0)),
            pl.BlockSpec((128, 128), lambda i: (0, 0)),
            pl.BlockSpec((1, 128), lambda i: (0, 0)),
        ],
        out_specs=pl.BlockSpec((_B, 128), lambda i: (i, 0)),
        out_shape=jax.ShapeDtypeStruct((_NP, 128), jnp.float32),
    )(a, hp, dis, W2, b1)


def _tc3(cacc, g, dis, b2):
    def body(c_ref, g_ref, dis_ref, b_ref, o_ref):
        t = c_ref[...] + g_ref[...]
        o_ref[...] = t * dis_ref[...] + b_ref[...]

    return pl.pallas_call(
        body,
        grid=(_NP // _B,),
        in_specs=[
            pl.BlockSpec((_B, 128), lambda i: (i, 0)),
            pl.BlockSpec((_B, 128), lambda i: (i, 0)),
            pl.BlockSpec((_B, 1), lambda i: (i, 0)),
            pl.BlockSpec((1, 128), lambda i: (0, 0)),
        ],
        out_specs=pl.BlockSpec((_B, 128), lambda i: (i, 0)),
        out_shape=jax.ShapeDtypeStruct((_NP, 128), jnp.float32),
    )(cacc, g, dis, b2)


def kernel(x, edge_index, W1, b1, W2, b2):
    src = edge_index[0]
    dst = edge_index[1]
    src_p = jnp.concatenate([src, jnp.zeros((_PAD,), jnp.int32)])
    dst_p = jnp.concatenate([dst, jnp.full((_PAD,), _IGN, jnp.int32)])
    ign = jnp.int32(_IGN)
    in0 = (dst_p >= 0) & (dst_p < _HALF)
    in1 = dst_p >= _HALF
    src0 = jnp.where(in0, src_p, ign).reshape(16, _CH, 128)
    dst0 = jnp.where(in0, dst_p, ign).reshape(16, _CH, 128)
    src1 = jnp.where(in1, src_p, ign).reshape(16, _CH, 128)
    dst1 = jnp.where(in1, dst_p - _HALF, ign).reshape(16, _CH, 128)
    src_h = jnp.concatenate([src0[None], src1[None]]).reshape(32, _CH, 128)
    dst_h = jnp.concatenate([dst0[None], dst1[None]]).reshape(32, _CH, 128)
    x_pad = jnp.pad(x, ((0, _NP - _N), (0, 0)))

    hist = _hist_to_col(_sc_hist(dst_h))
    hp, dis = _tc1(x_pad, hist, W1)
    a = _sc_agg(hp, src_h, dst_h)
    g = _tc2(a, hp, dis, W2, b1.reshape(1, 128))
    cacc = _sc_agg(g, src_h, dst_h)
    out = _tc3(cacc, g, dis, b2.reshape(1, 128))
    return out[:_N]


# speedup vs baseline: 28.2502x; 1.4378x over previous
import functools

import jax
import jax.numpy as jnp
from jax import lax
from jax.experimental import pallas as pl
from jax.experimental.pallas import tpu as pltpu
from jax.experimental.pallas import tpu_sc as plsc

_N = 10000
_NP = 10240
_HALF = _NP // 2
_E = 320000
_EP = 327680
_PAD = _EP - _E
_CH = 160
_NBUF = 4
_PH = 2
_CHP = _CH // _PH
_STRIPE = _HALF // 16
_B = 1024
_IGN = -1


def _mesh():
    return plsc.VectorSubcoreMesh(core_axis_name="c", subcore_axis_name="s")


def _sc_hist(dst_h):
    @functools.partial(
        pl.kernel,
        out_type=jax.ShapeDtypeStruct((_NP,), jnp.float32),
        mesh=_mesh(),
        scratch_types=[
            pltpu.VMEM((_CH, 128), jnp.int32),
            pltpu.VMEM((128,), jnp.float32),
            pltpu.VMEM((_STRIPE,), jnp.float32),
            pltpu.VMEM_SHARED((_HALF,), jnp.float32),
            pltpu.SemaphoreType.DMA,
        ],
    )
    def k(dst_ref, out_ref, didx, ones, zbuf, hist, sem):
        c = lax.axis_index("c")
        s = lax.axis_index("s")
        w = c * 16 + s
        ov = jnp.ones((16,), jnp.float32)
        zv = jnp.zeros((16,), jnp.float32)

        def fill(i, _):
            ones[pl.ds(i * 16, 16)] = ov
            return 0

        lax.fori_loop(0, 8, fill, 0)

        def zrow(i, _):
            zbuf[pl.ds(i * 16, 16)] = zv
            return 0

        lax.fori_loop(0, _STRIPE // 16, zrow, 0)
        pltpu.sync_copy(zbuf, hist.at[pl.ds(s * _STRIPE, _STRIPE)])
        pltpu.sync_copy(dst_ref.at[w], didx)
        plsc.subcore_barrier()

        def body(t, _):
            descs = []
            for b in range(8):
                descs.append(
                    pltpu.async_copy(
                        ones,
                        hist.at[
                            plsc.Indices(
                                didx.at[t * 8 + b], ignored_value=_IGN
                            )
                        ],
                        sem,
                        add=True,
                    )
                )
            for d in descs:
                d.wait()
            return 0

        lax.fori_loop(0, _CH // 8, body, 0)
        plsc.subcore_barrier()
        pltpu.sync_copy(hist.at[pl.ds(s * _STRIPE, _STRIPE)], zbuf)
        pltpu.sync_copy(
            zbuf, out_ref.at[pl.ds(c * _HALF + s * _STRIPE, _STRIPE)]
        )

    return k(dst_h)


def _hist_to_col(hist):
    return hist.reshape(_NP, 1)


def _sc_agg(table, src_h, dst_h):
    @functools.partial(
        pl.kernel,
        out_type=jax.ShapeDtypeStruct((_NP, 128), jnp.float32),
        mesh=_mesh(),
        scratch_types=[
            pltpu.VMEM((_CHP, 128), jnp.int32),
            pltpu.VMEM((_CHP, 128), jnp.int32),
            pltpu.VMEM((_NBUF, 128, 128), jnp.float32),
            pltpu.VMEM((20, 128), jnp.float32),
            pltpu.VMEM_SHARED((_HALF, 128), jnp.float32),
            pltpu.SemaphoreType.DMA,
            pltpu.SemaphoreType.DMA,
        ],
    )
    def k(table_ref, src_ref, dst_ref, out_ref, sidx, didx, bufs, zbuf,
          accum, gsem, ssem):
        c = lax.axis_index("c")
        s = lax.axis_index("s")
        w = c * 16 + s
        zv = jnp.zeros((16,), jnp.float32)

        def zrow(i, _):
            for q in range(8):
                zbuf[i, pl.ds(q * 16, 16)] = zv
            return 0

        lax.fori_loop(0, 20, zrow, 0)

        def zcp(t, _):
            pltpu.sync_copy(zbuf, accum.at[pl.ds(s * _STRIPE + t * 20, 20)])
            return 0

        lax.fori_loop(0, _STRIPE // 20, zcp, 0)
        plsc.subcore_barrier()

        def fire_gather(chunk, b):
            return pltpu.async_copy(
                table_ref.at[
                    plsc.Indices(sidx.at[chunk], ignored_value=_IGN)
                ],
                bufs.at[b],
                gsem,
            )

        def fire_scatter(chunk, b):
            return pltpu.async_copy(
                bufs.at[b],
                accum.at[
                    plsc.Indices(didx.at[chunk], ignored_value=_IGN)
                ],
                ssem,
                add=True,
            )

        def wait_gather(b):
            pltpu.make_async_copy(
                table_ref.at[
                    plsc.Indices(sidx.at[0], ignored_value=_IGN)
                ],
                bufs.at[b],
                gsem,
            ).wait()

        def wait_scatter(b):
            pltpu.make_async_copy(
                bufs.at[b],
                accum.at[
                    plsc.Indices(didx.at[0], ignored_value=_IGN)
                ],
                ssem,
            ).wait()

        def run_phase(p, _):
            pltpu.sync_copy(src_ref.at[w, p], sidx)
            pltpu.sync_copy(dst_ref.at[w, p], didx)
            for b in range(_NBUF):
                fire_gather(b, b)
            for b in range(_NBUF):
                wait_gather(b)
                fire_scatter(b, b)

            def body(t, _):
                base = t * _NBUF
                for b in range(_NBUF):
                    wait_scatter(b)
                    fire_gather(base + b, b)
                for b in range(_NBUF):
                    wait_gather(b)
                    fire_scatter(base + b, b)
                return 0

            lax.fori_loop(1, _CHP // _NBUF, body, 0)
            for b in range(_NBUF):
                wait_scatter(b)
            return 0

        lax.fori_loop(0, _PH, run_phase, 0)
        plsc.subcore_barrier()
        pltpu.sync_copy(
            accum.at[pl.ds(s * _STRIPE, _STRIPE)],
            out_ref.at[pl.ds(c * _HALF + s * _STRIPE, _STRIPE)],
        )

    return k(table, src_h, dst_h)


def _tc1(x_pad, hist, W1):
    def body(x_ref, h_ref, w_ref, hp_ref, dis_ref):
        deg = 1.0 + h_ref[...]
        dis = lax.rsqrt(deg)
        hp_ref[...] = jnp.dot(x_ref[...], w_ref[...],
                              preferred_element_type=jnp.float32) * dis
        dis_ref[...] = dis

    return pl.pallas_call(
        body,
        grid=(_NP // _B,),
        in_specs=[
            pl.BlockSpec((_B, 128), lambda i: (i, 0)),
            pl.BlockSpec((_B, 1), lambda i: (i, 0)),
            pl.BlockSpec((128, 128), lambda i: (0, 0)),
        ],
        out_specs=[
            pl.BlockSpec((_B, 128), lambda i: (i, 0)),
            pl.BlockSpec((_B, 1), lambda i: (i, 0)),
        ],
        out_shape=[
            jax.ShapeDtypeStruct((_NP, 128), jnp.float32),
            jax.ShapeDtypeStruct((_NP, 1), jnp.float32),
        ],
    )(x_pad, hist, W1)


def _tc2(a, hp, dis, W2, b1):
    def body(a_ref, h_ref, dis_ref, w_ref, b_ref, g_ref):
        t = a_ref[...] + h_ref[...]
        dis = dis_ref[...]
        z = t * dis + b_ref[...]
        h2 = jnp.maximum(z, 0.0)
        g_ref[...] = jnp.dot(h2, w_ref[...],
                             preferred_element_type=jnp.float32) * dis

    return pl.pallas_call(
        body,
        grid=(_NP // _B,),
        in_specs=[
            pl.BlockSpec((_B, 128), lambda i: (i, 0)),
            pl.BlockSpec((_B, 128), lambda i: (i, 0)),
            pl.BlockSpec((_B, 1), lambda i: (i, 0)),
            pl.BlockSpec((128, 128), lambda i: (0, 0)),
            pl.BlockSpec((1, 128), lambda i: (0, 0)),
        ],
        out_specs=pl.BlockSpec((_B, 128), lambda i: (i, 0)),
        out_shape=jax.ShapeDtypeStruct((_NP, 128), jnp.float32),
    )(a, hp, dis, W2, b1)


def _tc3(cacc, g, dis, b2):
    def body(c_ref, g_ref, dis_ref, b_ref, o_ref):
        t = c_ref[...] + g_ref[...]
        o_ref[...] = t * dis_ref[...] + b_ref[...]

    return pl.pallas_call(
        body,
        grid=(_NP // _B,),
        in_specs=[
            pl.BlockSpec((_B, 128), lambda i: (i, 0)),
            pl.BlockSpec((_B, 128), lambda i: (i, 0)),
            pl.BlockSpec((_B, 1), lambda i: (i, 0)),
            pl.BlockSpec((1, 128), lambda i: (0, 0)),
        ],
        out_specs=pl.BlockSpec((_B, 128), lambda i: (i, 0)),
        out_shape=jax.ShapeDtypeStruct((_NP, 128), jnp.float32),
    )(cacc, g, dis, b2)


def kernel(x, edge_index, W1, b1, W2, b2):
    src = edge_index[0]
    dst = edge_index[1]
    src_p = jnp.concatenate([src, jnp.zeros((_PAD,), jnp.int32)])
    dst_p = jnp.concatenate([dst, jnp.full((_PAD,), _IGN, jnp.int32)])
    ign = jnp.int32(_IGN)
    in0 = (dst_p >= 0) & (dst_p < _HALF)
    in1 = dst_p >= _HALF
    src0 = jnp.where(in0, src_p, ign).reshape(16, _CH, 128)
    dst0 = jnp.where(in0, dst_p, ign).reshape(16, _CH, 128)
    src1 = jnp.where(in1, src_p, ign).reshape(16, _CH, 128)
    dst1 = jnp.where(in1, dst_p - _HALF, ign).reshape(16, _CH, 128)
    src_h = jnp.concatenate([src0[None], src1[None]]).reshape(32, _CH, 128)
    dst_h = jnp.concatenate([dst0[None], dst1[None]]).reshape(32, _CH, 128)
    src_h4 = src_h.reshape(32, _PH, _CHP, 128)
    dst_h4 = dst_h.reshape(32, _PH, _CHP, 128)
    x_pad = jnp.pad(x, ((0, _NP - _N), (0, 0)))

    hist = _hist_to_col(_sc_hist(dst_h))
    hp, dis = _tc1(x_pad, hist, W1)
    a = _sc_agg(hp, src_h4, dst_h4)
    g = _tc2(a, hp, dis, W2, b1.reshape(1, 128))
    cacc = _sc_agg(g, src_h4, dst_h4)
    out = _tc3(cacc, g, dis, b2.reshape(1, 128))
    return out[:_N]


# speedup vs baseline: 28.8737x; 1.0221x over previous
import functools

import jax
import jax.numpy as jnp
from jax import lax
from jax.experimental import pallas as pl
from jax.experimental.pallas import tpu as pltpu
from jax.experimental.pallas import tpu_sc as plsc

_N = 10000
_NP = 10240
_HALF = _NP // 2
_E = 320000
_EP = 327680
_PAD = _EP - _E
_CH = 160
_NBUF = 4
_PH = 2
_CHP = _CH // _PH
_STRIPE = _HALF // 16
_B = 2048
_IGN = -1


def _mesh():
    return plsc.VectorSubcoreMesh(core_axis_name="c", subcore_axis_name="s")


def _sc_hist(dst_h):
    @functools.partial(
        pl.kernel,
        out_type=jax.ShapeDtypeStruct((_NP,), jnp.float32),
        mesh=_mesh(),
        scratch_types=[
            pltpu.VMEM((_CH, 128), jnp.int32),
            pltpu.VMEM((128,), jnp.float32),
            pltpu.VMEM((_STRIPE,), jnp.float32),
            pltpu.VMEM_SHARED((_HALF,), jnp.float32),
            pltpu.SemaphoreType.DMA,
        ],
    )
    def k(dst_ref, out_ref, didx, ones, zbuf, hist, sem):
        c = lax.axis_index("c")
        s = lax.axis_index("s")
        w = c * 16 + s
        ov = jnp.ones((16,), jnp.float32)
        zv = jnp.zeros((16,), jnp.float32)

        def fill(i, _):
            ones[pl.ds(i * 16, 16)] = ov
            return 0

        lax.fori_loop(0, 8, fill, 0)

        def zrow(i, _):
            zbuf[pl.ds(i * 16, 16)] = zv
            return 0

        lax.fori_loop(0, _STRIPE // 16, zrow, 0)
        pltpu.sync_copy(zbuf, hist.at[pl.ds(s * _STRIPE, _STRIPE)])
        pltpu.sync_copy(dst_ref.at[w], didx)
        plsc.subcore_barrier()

        def body(t, _):
            descs = []
            for b in range(16):
                descs.append(
                    pltpu.async_copy(
                        ones,
                        hist.at[
                            plsc.Indices(
                                didx.at[t * 16 + b], ignored_value=_IGN
                            )
                        ],
                        sem,
                        add=True,
                    )
                )
            for d in descs:
                d.wait()
            return 0

        lax.fori_loop(0, _CH // 16, body, 0)
        plsc.subcore_barrier()
        pltpu.sync_copy(hist.at[pl.ds(s * _STRIPE, _STRIPE)], zbuf)
        pltpu.sync_copy(
            zbuf, out_ref.at[pl.ds(c * _HALF + s * _STRIPE, _STRIPE)]
        )

    return k(dst_h)


def _hist_to_col(hist):
    return hist.reshape(_NP, 1)


def _sc_agg(table, src_h, dst_h):
    @functools.partial(
        pl.kernel,
        out_type=jax.ShapeDtypeStruct((_NP, 128), jnp.float32),
        mesh=_mesh(),
        scratch_types=[
            pltpu.VMEM((_CHP, 128), jnp.int32),
            pltpu.VMEM((_CHP, 128), jnp.int32),
            pltpu.VMEM((_NBUF, 128, 128), jnp.float32),
            pltpu.VMEM((20, 128), jnp.float32),
            pltpu.VMEM_SHARED((_HALF, 128), jnp.float32),
            pltpu.SemaphoreType.DMA,
            pltpu.SemaphoreType.DMA,
        ],
    )
    def k(table_ref, src_ref, dst_ref, out_ref, sidx, didx, bufs, zbuf,
          accum, gsem, ssem):
        c = lax.axis_index("c")
        s = lax.axis_index("s")
        w = c * 16 + s
        zv = jnp.zeros((16,), jnp.float32)

        def zrow(i, _):
            for q in range(8):
                zbuf[i, pl.ds(q * 16, 16)] = zv
            return 0

        lax.fori_loop(0, 20, zrow, 0)

        def zcp(t, _):
            pltpu.sync_copy(zbuf, accum.at[pl.ds(s * _STRIPE + t * 20, 20)])
            return 0

        lax.fori_loop(0, _STRIPE // 20, zcp, 0)
        plsc.subcore_barrier()

        def fire_gather(chunk, b):
            return pltpu.async_copy(
                table_ref.at[
                    plsc.Indices(sidx.at[chunk], ignored_value=_IGN)
                ],
                bufs.at[b],
                gsem,
            )

        def fire_scatter(chunk, b):
            return pltpu.async_copy(
                bufs.at[b],
                accum.at[
                    plsc.Indices(didx.at[chunk], ignored_value=_IGN)
                ],
                ssem,
                add=True,
            )

        def wait_gather(b):
            pltpu.make_async_copy(
                table_ref.at[
                    plsc.Indices(sidx.at[0], ignored_value=_IGN)
                ],
                bufs.at[b],
                gsem,
            ).wait()

        def wait_scatter(b):
            pltpu.make_async_copy(
                bufs.at[b],
                accum.at[
                    plsc.Indices(didx.at[0], ignored_value=_IGN)
                ],
                ssem,
            ).wait()

        def run_phase(p, _):
            pltpu.sync_copy(src_ref.at[w, p], sidx)
            pltpu.sync_copy(dst_ref.at[w, p], didx)
            for b in range(_NBUF):
                fire_gather(b, b)
            for b in range(_NBUF):
                wait_gather(b)
                fire_scatter(b, b)

            def body(t, _):
                base = t * _NBUF
                for b in range(_NBUF):
                    wait_scatter(b)
                    fire_gather(base + b, b)
                for b in range(_NBUF):
                    wait_gather(b)
                    fire_scatter(base + b, b)
                return 0

            lax.fori_loop(1, _CHP // _NBUF, body, 0)
            for b in range(_NBUF):
                wait_scatter(b)
            return 0

        lax.fori_loop(0, _PH, run_phase, 0)
        plsc.subcore_barrier()
        pltpu.sync_copy(
            accum.at[pl.ds(s * _STRIPE, _STRIPE)],
            out_ref.at[pl.ds(c * _HALF + s * _STRIPE, _STRIPE)],
        )

    return k(table, src_h, dst_h)


def _tc1a(x_pad, W1):
    def body(x_ref, w_ref, m_ref):
        m_ref[...] = jnp.dot(x_ref[...], w_ref[...],
                             preferred_element_type=jnp.float32)

    return pl.pallas_call(
        body,
        grid=(_NP // _B,),
        in_specs=[
            pl.BlockSpec((_B, 128), lambda i: (i, 0)),
            pl.BlockSpec((128, 128), lambda i: (0, 0)),
        ],
        out_specs=pl.BlockSpec((_B, 128), lambda i: (i, 0)),
        out_shape=jax.ShapeDtypeStruct((_NP, 128), jnp.float32),
    )(x_pad, W1)


def _tc1b(m, hist):
    def body(m_ref, h_ref, hp_ref, dis_ref):
        deg = 1.0 + h_ref[...]
        dis = lax.rsqrt(deg)
        hp_ref[...] = m_ref[...] * dis
        dis_ref[...] = dis

    return pl.pallas_call(
        body,
        grid=(_NP // _B,),
        in_specs=[
            pl.BlockSpec((_B, 128), lambda i: (i, 0)),
            pl.BlockSpec((_B, 1), lambda i: (i, 0)),
        ],
        out_specs=[
            pl.BlockSpec((_B, 128), lambda i: (i, 0)),
            pl.BlockSpec((_B, 1), lambda i: (i, 0)),
        ],
        out_shape=[
            jax.ShapeDtypeStruct((_NP, 128), jnp.float32),
            jax.ShapeDtypeStruct((_NP, 1), jnp.float32),
        ],
    )(m, hist)


def _tc2(a, hp, dis, W2, b1):
    def body(a_ref, h_ref, dis_ref, w_ref, b_ref, g_ref):
        t = a_ref[...] + h_ref[...]
        dis = dis_ref[...]
        z = t * dis + b_ref[...]
        h2 = jnp.maximum(z, 0.0)
        g_ref[...] = jnp.dot(h2, w_ref[...],
                             preferred_element_type=jnp.float32) * dis

    return pl.pallas_call(
        body,
        grid=(_NP // _B,),
        in_specs=[
            pl.BlockSpec((_B, 128), lambda i: (i, 0)),
            pl.BlockSpec((_B, 128), lambda i: (i, 0)),
            pl.BlockSpec((_B, 1), lambda i: (i, 0)),
            pl.BlockSpec((128, 128), lambda i: (0, 0)),
            pl.BlockSpec((1, 128), lambda i: (0, 0)),
        ],
        out_specs=pl.BlockSpec((_B, 128), lambda i: (i, 0)),
        out_shape=jax.ShapeDtypeStruct((_NP, 128), jnp.float32),
    )(a, hp, dis, W2, b1)


def _tc3(cacc, g, dis, b2):
    def body(c_ref, g_ref, dis_ref, b_ref, o_ref):
        t = c_ref[...] + g_ref[...]
        o_ref[...] = t * dis_ref[...] + b_ref[...]

    bo = 1000
    return pl.pallas_call(
        body,
        grid=(_N // bo,),
        in_specs=[
            pl.BlockSpec((bo, 128), lambda i: (i, 0)),
            pl.BlockSpec((bo, 128), lambda i: (i, 0)),
            pl.BlockSpec((bo, 1), lambda i: (i, 0)),
            pl.BlockSpec((1, 128), lambda i: (0, 0)),
        ],
        out_specs=pl.BlockSpec((bo, 128), lambda i: (i, 0)),
        out_shape=jax.ShapeDtypeStruct((_N, 128), jnp.float32),
    )(cacc, g, dis, b2)


def kernel(x, edge_index, W1, b1, W2, b2):
    src = edge_index[0]
    dst = edge_index[1]
    src_p = jnp.concatenate([src, jnp.zeros((_PAD,), jnp.int32)])
    dst_p = jnp.concatenate([dst, jnp.full((_PAD,), _IGN, jnp.int32)])
    ign = jnp.int32(_IGN)
    in0 = (dst_p >= 0) & (dst_p < _HALF)
    in1 = dst_p >= _HALF
    src0 = jnp.where(in0, src_p, ign).reshape(16, _CH, 128)
    dst0 = jnp.where(in0, dst_p, ign).reshape(16, _CH, 128)
    src1 = jnp.where(in1, src_p, ign).reshape(16, _CH, 128)
    dst1 = jnp.where(in1, dst_p - _HALF, ign).reshape(16, _CH, 128)
    src_h = jnp.concatenate([src0[None], src1[None]]).reshape(32, _CH, 128)
    dst_h = jnp.concatenate([dst0[None], dst1[None]]).reshape(32, _CH, 128)
    src_h4 = src_h.reshape(32, _PH, _CHP, 128)
    dst_h4 = dst_h.reshape(32, _PH, _CHP, 128)
    x_pad = jnp.pad(x, ((0, _NP - _N), (0, 0)))

    m1 = _tc1a(x_pad, W1)
    hist = _hist_to_col(_sc_hist(dst_h))
    hp, dis = _tc1b(m1, hist)
    a = _sc_agg(hp, src_h4, dst_h4)
    g = _tc2(a, hp, dis, W2, b1.reshape(1, 128))
    cacc = _sc_agg(g, src_h4, dst_h4)
    return _tc3(cacc, g, dis, b2.reshape(1, 128))
